# baseline (speedup 1.0000x reference)
import jax
import jax.numpy as jnp
from jax.experimental import pallas as pl
from jax.experimental.pallas import tpu as pltpu


def _round_up(x, m):
    return (x + m - 1) // m * m


def _fused_kernel(f0_ref, f1_ref, f2_ref, w01_ref, w02a_ref, w02b_ref,
                  w12_ref, wf_ref, bias_ref, out_ref):
    D = w02a_ref.shape[0]
    f0 = f0_ref[...]
    f1 = f1_ref[...]
    f2 = f2_ref[...]

    r01 = jnp.dot(f0, w01_ref[:D], preferred_element_type=jnp.float32)
    r01 = r01 + jnp.dot(f1, w01_ref[D:], preferred_element_type=jnp.float32)
    r12 = jnp.dot(f1, w12_ref[:D], preferred_element_type=jnp.float32)
    r12 = r12 + jnp.dot(f2, w12_ref[D:], preferred_element_type=jnp.float32)
    r02 = jnp.dot(f0, w02a_ref[...], preferred_element_type=jnp.float32)
    r02 = r02 + jnp.dot(f2, w02b_ref[...],
                        preferred_element_type=jnp.float32)

    wf = wf_ref[...]
    out = jnp.dot(r01, wf[:D], preferred_element_type=jnp.float32)
    out = out + jnp.dot(r02, wf[D:2 * D], preferred_element_type=jnp.float32)
    out = out + jnp.dot(r12, wf[2 * D:], preferred_element_type=jnp.float32)
    out_ref[...] = out + bias_ref[...]


def kernel(features, wd, bd, wf, bf):
    B, three, D = features.shape
    NF = wf.shape[1]

    w01 = wd[:2 * D, :D].astype(jnp.bfloat16)
    w12 = wd[D:, 2 * D:].astype(jnp.bfloat16)
    w02a = wd[:D, D:2 * D].astype(jnp.bfloat16)
    w02b = wd[2 * D:, D:2 * D].astype(jnp.bfloat16)
    bias2 = jnp.dot(bd, wf, preferred_element_type=jnp.float32) + bf

    f0 = features[:, 0, :].astype(jnp.bfloat16)
    f1 = features[:, 1, :].astype(jnp.bfloat16)
    f2 = features[:, 2, :].astype(jnp.bfloat16)

    TB = 1024
    B_pad = _round_up(B, 2 * TB)
    if B_pad != B:
        pad = ((0, B_pad - B), (0, 0))
        f0 = jnp.pad(f0, pad)
        f1 = jnp.pad(f1, pad)
        f2 = jnp.pad(f2, pad)
    n_inner = B_pad // TB // 2

    compiler_params = pltpu.CompilerParams(
        dimension_semantics=("parallel", "arbitrary"),
        vmem_limit_bytes=64 * 1024 * 1024,
    )

    def _tile(c, j, n=n_inner):
        return (c * n + j, 0)

    def _whole(c, j):
        return (0, 0)

    out_pad = pl.pallas_call(
        _fused_kernel,
        out_shape=jax.ShapeDtypeStruct((B_pad, NF), jnp.float32),
        grid=(2, n_inner),
        in_specs=[
            pl.BlockSpec((TB, D), _tile),
            pl.BlockSpec((TB, D), _tile),
            pl.BlockSpec((TB, D), _tile),
            pl.BlockSpec((2 * D, D), _whole),
            pl.BlockSpec((D, D), _whole),
            pl.BlockSpec((D, D), _whole),
            pl.BlockSpec((2 * D, D), _whole),
            pl.BlockSpec((3 * D, NF), _whole),
            pl.BlockSpec((1, NF), _whole),
        ],
        out_specs=pl.BlockSpec((TB, NF), _tile),
        compiler_params=compiler_params,
    )(f0, f1, f2, w01, w02a, w02b, w12, wf, bias2)

    return out_pad[:B, :6].astype(features.dtype)


# speedup vs baseline: 1.0143x; 1.0143x over previous
import jax
import jax.numpy as jnp
from jax.experimental import pallas as pl
from jax.experimental.pallas import tpu as pltpu


def _round_up(x, m):
    return (x + m - 1) // m * m


def _make_kernel(D):
    def _fused_kernel(f0_ref, f1_ref, f2_ref, wd_hbm, wf_ref, bias_ref,
                      out_ref, w01f, w02f, w12f, w01s, w02s, w12s, sems):
        j = pl.program_id(1)

        @pl.when(j == 0)
        def _prep_weights():
            blocks = [
                (0, 0, w01f, 0), (1, 0, w01f, 1),
                (0, 1, w02f, 0), (2, 1, w02f, 1),
                (1, 2, w12f, 0), (2, 2, w12f, 1),
            ]
            for n, (br, bc, dst, half) in enumerate(blocks):
                pltpu.make_async_copy(
                    wd_hbm.at[pl.ds(br * D, D), pl.ds(bc * D, D)],
                    dst.at[pl.ds(half * D, D)],
                    sems.at[n],
                ).start()
            for n, (br, bc, dst, half) in enumerate(blocks):
                pltpu.make_async_copy(
                    dst.at[pl.ds(half * D, D)],
                    dst.at[pl.ds(half * D, D)],
                    sems.at[n],
                ).wait()
            w01s[...] = w01f[...].astype(jnp.bfloat16)
            w02s[...] = w02f[...].astype(jnp.bfloat16)
            w12s[...] = w12f[...].astype(jnp.bfloat16)

        f0 = f0_ref[...]
        f1 = f1_ref[...]
        f2 = f2_ref[...]

        r01 = jnp.dot(f0, w01s[:D], preferred_element_type=jnp.float32)
        r01 = r01 + jnp.dot(f1, w01s[D:], preferred_element_type=jnp.float32)
        r12 = jnp.dot(f1, w12s[:D], preferred_element_type=jnp.float32)
        r12 = r12 + jnp.dot(f2, w12s[D:], preferred_element_type=jnp.float32)
        r02 = jnp.dot(f0, w02s[:D], preferred_element_type=jnp.float32)
        r02 = r02 + jnp.dot(f2, w02s[D:], preferred_element_type=jnp.float32)

        wf = wf_ref[...]
        out = jnp.dot(r01, wf[:D], preferred_element_type=jnp.float32)
        out = out + jnp.dot(r02, wf[D:2 * D],
                            preferred_element_type=jnp.float32)
        out = out + jnp.dot(r12, wf[2 * D:],
                            preferred_element_type=jnp.float32)
        out_ref[...] = out + bias_ref[...]

    return _fused_kernel


def kernel(features, wd, bd, wf, bf):
    B, three, D = features.shape
    NF = wf.shape[1]

    bias2 = jnp.dot(bd, wf, preferred_element_type=jnp.float32) + bf
    f0 = features[:, 0, :].astype(jnp.bfloat16)
    f1 = features[:, 1, :].astype(jnp.bfloat16)
    f2 = features[:, 2, :].astype(jnp.bfloat16)

    TB = 512
    B_pad = _round_up(B, 2 * TB)
    if B_pad != B:
        pad = ((0, B_pad - B), (0, 0))
        f0 = jnp.pad(f0, pad)
        f1 = jnp.pad(f1, pad)
        f2 = jnp.pad(f2, pad)
    n_inner = B_pad // TB // 2

    compiler_params = pltpu.CompilerParams(
        dimension_semantics=("parallel", "arbitrary"),
        vmem_limit_bytes=64 * 1024 * 1024,
    )

    def _tile(c, j, n=n_inner):
        return (c * n + j, 0)

    def _whole(c, j):
        return (0, 0)

    out_pad = pl.pallas_call(
        _make_kernel(D),
        out_shape=jax.ShapeDtypeStruct((B_pad, NF), jnp.float32),
        grid=(2, n_inner),
        in_specs=[
            pl.BlockSpec((TB, D), _tile),
            pl.BlockSpec((TB, D), _tile),
            pl.BlockSpec((TB, D), _tile),
            pl.BlockSpec(memory_space=pl.ANY),
            pl.BlockSpec((3 * D, NF), _whole),
            pl.BlockSpec((1, NF), _whole),
        ],
        out_specs=pl.BlockSpec((TB, NF), _tile),
        scratch_shapes=[
            pltpu.VMEM((2 * D, D), jnp.float32),
            pltpu.VMEM((2 * D, D), jnp.float32),
            pltpu.VMEM((2 * D, D), jnp.float32),
            pltpu.VMEM((2 * D, D), jnp.bfloat16),
            pltpu.VMEM((2 * D, D), jnp.bfloat16),
            pltpu.VMEM((2 * D, D), jnp.bfloat16),
            pltpu.SemaphoreType.DMA((6,)),
        ],
        compiler_params=compiler_params,
    )(f0, f1, f2, wd, wf, bias2)

    return out_pad[:B, :6].astype(features.dtype)


# speedup vs baseline: 1.1583x; 1.1420x over previous
import jax
import jax.numpy as jnp
from jax.experimental import pallas as pl
from jax.experimental.pallas import tpu as pltpu


def _round_up(x, m):
    return (x + m - 1) // m * m


def _fused_kernel(ft_ref, w01_ref, w02a_ref, w02b_ref, w12_ref,
                  wf_ref, bias_ref, out_ref):
    D = w02a_ref.shape[0]
    f0 = ft_ref[0]
    f1 = ft_ref[1]
    f2 = ft_ref[2]

    r01 = jnp.dot(f0, w01_ref[:D], preferred_element_type=jnp.float32)
    r01 = r01 + jnp.dot(f1, w01_ref[D:], preferred_element_type=jnp.float32)
    r12 = jnp.dot(f1, w12_ref[:D], preferred_element_type=jnp.float32)
    r12 = r12 + jnp.dot(f2, w12_ref[D:], preferred_element_type=jnp.float32)
    r02 = jnp.dot(f0, w02a_ref[...], preferred_element_type=jnp.float32)
    r02 = r02 + jnp.dot(f2, w02b_ref[...],
                        preferred_element_type=jnp.float32)

    wf = wf_ref[...]
    out = jnp.dot(r01, wf[:D], preferred_element_type=jnp.float32)
    out = out + jnp.dot(r02, wf[D:2 * D], preferred_element_type=jnp.float32)
    out = out + jnp.dot(r12, wf[2 * D:], preferred_element_type=jnp.float32)
    out_ref[...] = out + bias_ref[...]


def kernel(features, wd, bd, wf, bf):
    B, three, D = features.shape
    NF = wf.shape[1]

    w01 = wd[:2 * D, :D].astype(jnp.bfloat16)
    w12 = wd[D:, 2 * D:].astype(jnp.bfloat16)
    w02a = wd[:D, D:2 * D].astype(jnp.bfloat16)
    w02b = wd[2 * D:, D:2 * D].astype(jnp.bfloat16)
    bias2 = jnp.dot(bd, wf, preferred_element_type=jnp.float32) + bf

    ft = features.transpose((1, 0, 2)).astype(jnp.bfloat16)

    TB = 1024
    B_pad = _round_up(B, 2 * TB)
    if B_pad != B:
        ft = jnp.pad(ft, ((0, 0), (0, B_pad - B), (0, 0)))
    n_inner = B_pad // TB // 2

    compiler_params = pltpu.CompilerParams(
        dimension_semantics=("parallel", "arbitrary"),
        vmem_limit_bytes=64 * 1024 * 1024,
    )

    def _tile3(c, j, n=n_inner):
        return (0, c * n + j, 0)

    def _tile(c, j, n=n_inner):
        return (c * n + j, 0)

    def _whole(c, j):
        return (0, 0)

    out_pad = pl.pallas_call(
        _fused_kernel,
        out_shape=jax.ShapeDtypeStruct((B_pad, NF), jnp.float32),
        grid=(2, n_inner),
        in_specs=[
            pl.BlockSpec((3, TB, D), _tile3),
            pl.BlockSpec((2 * D, D), _whole),
            pl.BlockSpec((D, D), _whole),
            pl.BlockSpec((D, D), _whole),
            pl.BlockSpec((2 * D, D), _whole),
            pl.BlockSpec((3 * D, NF), _whole),
            pl.BlockSpec((1, NF), _whole),
        ],
        out_specs=pl.BlockSpec((TB, NF), _tile),
        compiler_params=compiler_params,
    )(ft, w01, w02a, w02b, w12, wf, bias2)

    return out_pad[:B, :6].astype(features.dtype)
